# Initial kernel scaffold; baseline (speedup 1.0000x reference)
import jax
import jax.numpy as jnp
from jax.experimental import pallas as pl


def kernel(node_features, edge_index, edge_features, W_lin, b_lin, W1, b1, W2, b2, W3, b3, W4, b4, W_cls, b_cls):
    raise NotImplementedError("write your pallas kernel here")


# speedup vs baseline: 1.0198x; 1.0198x over previous
import functools

import jax
import jax.numpy as jnp
from jax import lax
from jax.experimental import pallas as pl
from jax.experimental.pallas import tpu as pltpu
from jax.experimental.pallas import tpu_sc as plsc

N_N = 10000
N_E = 160000
EB = 80
NB = 125
CH = 40
NCH = N_N // CH
F32 = jnp.float32

_MESH = plsc.VectorSubcoreMesh(core_axis_name="c", subcore_axis_name="s")


def _chunked(sid, fn):
    for z in range(16):
        ck = sid + 16 * z
        @pl.when(ck < NCH)
        def _(ck=ck):
            fn(ck)


def _deg_body(src_hbm, dst_hbm, ones_hbm, zeros_hbm, out0, out1,
              idx, ones_v, zer, wb, deg_sh):
    cid = lax.axis_index("c")
    sid = lax.axis_index("s")
    pltpu.sync_copy(ones_hbm, ones_v)
    pltpu.sync_copy(zeros_hbm, zer)
    for side in range(2):
        @pl.when(cid == side)
        def _(side=side):
            e_hbm = src_hbm if side == 0 else dst_hbm
            out_hbm = out0 if side == 0 else out1
            _chunked(sid, lambda ck: pltpu.sync_copy(
                zer, deg_sh.at[pl.ds(ck * CH, CH)]))
            plsc.subcore_barrier()

            def step(j, carry):
                base = j * EB
                pltpu.sync_copy(e_hbm.at[pl.ds(base, EB)], idx)
                pltpu.sync_copy(ones_v, deg_sh.at[idx], add=True)
                return carry

            @pl.when(sid == 0)
            def _st():
                lax.fori_loop(0, N_E // EB, step, 0)
            plsc.subcore_barrier()

            def wback(ck):
                sl = pl.ds(ck * CH, CH)
                pltpu.sync_copy(deg_sh.at[sl], wb)
                pltpu.sync_copy(wb, out_hbm.at[sl])

            _chunked(sid, wback)


def _deg_call(src1, dst1, ones128, zeros128):
    f = pl.kernel(
        _deg_body,
        out_type=[jax.ShapeDtypeStruct((N_N, 128), F32)] * 2,
        mesh=_MESH,
        scratch_types=[
            pltpu.VMEM((EB,), jnp.int32),
            pltpu.VMEM((EB, 128), F32),
            pltpu.VMEM((CH, 128), F32),
            pltpu.VMEM((CH, 128), F32),
            pltpu.VMEM_SHARED((N_N, 128), F32),
        ],
    )
    return f(src1, dst1, ones128, zeros128)


def _make_spmm(n_chunks_per_core, width):
    C, W = n_chunks_per_core, width
    n_tab = 2 * C

    def body(src_hbm, dst_hbm, zeros_hbm, *rest):
        h_refs = rest[:n_tab]
        out_refs = rest[n_tab:2 * n_tab]
        sidx, didx, rows, zer, wb, agg, sem = rest[2 * n_tab:]
        cid = lax.axis_index("c")
        sid = lax.axis_index("s")
        pltpu.sync_copy(zeros_hbm, zer)
        for side in range(2):
            @pl.when(cid == side)
            def _(side=side):
                for k in range(C):
                    h_hbm = h_refs[side * C + k]
                    out_hbm = out_refs[side * C + k]
                    _chunked(sid, lambda ck: pltpu.sync_copy(
                        zer, agg.at[pl.ds(ck * CH, CH)]))
                    plsc.subcore_barrier()

                    def step(j, carry):
                        base = sid * (NB * EB) + j * EB
                        pltpu.sync_copy(src_hbm.at[pl.ds(base, EB)], sidx)
                        pltpu.sync_copy(dst_hbm.at[pl.ds(base, EB)], didx)
                        pltpu.async_copy(h_hbm.at[sidx], rows, sem).wait()
                        pltpu.sync_copy(rows, agg.at[didx], add=True)
                        return carry

                    lax.fori_loop(0, NB, step, 0)
                    plsc.subcore_barrier()

                    def wback(ck, out_hbm=out_hbm):
                        sl = pl.ds(ck * CH, CH)
                        pltpu.sync_copy(agg.at[sl], wb)
                        pltpu.sync_copy(wb, out_hbm.at[sl])

                    _chunked(sid, wback)
                    plsc.subcore_barrier()

    def call(src1, dst1, zeros_buf, h_list):
        f = pl.kernel(
            body,
            out_type=[jax.ShapeDtypeStruct((N_N, W), F32)] * n_tab,
            mesh=_MESH,
            scratch_types=[
                pltpu.VMEM((EB,), jnp.int32),
                pltpu.VMEM((EB,), jnp.int32),
                pltpu.VMEM((EB, W), F32),
                pltpu.VMEM((CH, W), F32),
                pltpu.VMEM((CH, W), F32),
                pltpu.VMEM_SHARED((N_N, W), F32),
                pltpu.SemaphoreType.DMA,
            ],
        )
        return f(src1, dst1, zeros_buf, *h_list)

    return call


_spmm_128x2 = _make_spmm(2, 128)
_spmm_128x1 = _make_spmm(1, 128)


EB3 = 40
NB3 = 125


def _spmm_split_body(src_hbm, dst_hbm, zeros_hbm, h_hbm, out0, out1,
                     sidx, didx, rows, zer, wb, agg, sem):
    cid = lax.axis_index("c")
    sid = lax.axis_index("s")
    pltpu.sync_copy(zeros_hbm, zer)
    for side in range(2):
        @pl.when(cid == side)
        def _(side=side):
            out_hbm = out0 if side == 0 else out1
            _chunked(sid, lambda ck: pltpu.sync_copy(
                zer, agg.at[pl.ds(ck * CH, CH)]))
            plsc.subcore_barrier()

            def step(j, carry):
                base = side * (N_E // 2) + sid * (NB3 * EB3) + j * EB3
                pltpu.sync_copy(src_hbm.at[pl.ds(base, EB3)], sidx)
                pltpu.sync_copy(dst_hbm.at[pl.ds(base, EB3)], didx)
                pltpu.async_copy(h_hbm.at[sidx], rows, sem).wait()
                pltpu.sync_copy(rows, agg.at[didx], add=True)
                return carry

            lax.fori_loop(0, NB3, step, 0)
            plsc.subcore_barrier()

            def wback(ck, out_hbm=out_hbm):
                sl = pl.ds(ck * CH, CH)
                pltpu.sync_copy(agg.at[sl], wb)
                pltpu.sync_copy(wb, out_hbm.at[sl])

            _chunked(sid, wback)


def _spmm_split(src1, dst1, zeros_buf, h):
    f = pl.kernel(
        _spmm_split_body,
        out_type=[jax.ShapeDtypeStruct((N_N, 128), F32)] * 2,
        mesh=_MESH,
        scratch_types=[
            pltpu.VMEM((EB3,), jnp.int32),
            pltpu.VMEM((EB3,), jnp.int32),
            pltpu.VMEM((EB3, 128), F32),
            pltpu.VMEM((CH, 128), F32),
            pltpu.VMEM((CH, 128), F32),
            pltpu.VMEM_SHARED((N_N, 128), F32),
            pltpu.SemaphoreType.DMA,
        ],
    )
    return f(src1, dst1, zeros_buf, h)


EB2 = 40
NB2 = 125


def _edge_body(hs_hbm, hd_hbm, src_hbm, dst_hbm, out_hbm,
               sidx, didx, a_v, b_v, o_v, sem_a, sem_b):
    cid = lax.axis_index("c")
    sid = lax.axis_index("s")
    wid = sid * 2 + cid

    def step(j, carry):
        base = wid * (NB2 * EB2) + j * EB2
        pltpu.sync_copy(src_hbm.at[pl.ds(base, EB2)], sidx)
        pltpu.sync_copy(dst_hbm.at[pl.ds(base, EB2)], didx)
        da = pltpu.async_copy(hs_hbm.at[sidx], a_v, sem_a)
        db = pltpu.async_copy(hd_hbm.at[didx], b_v, sem_b)
        da.wait()
        db.wait()
        for r in range(EB2):
            o_v[r, :] = a_v[r, pl.ds(0, 16)] + b_v[r, pl.ds(0, 16)]
        pltpu.sync_copy(o_v, out_hbm.at[pl.ds(wid * 5000 + j * EB2, EB2)])
        return carry

    lax.fori_loop(0, NB2, step, 0)


def _edge_call(hs, hd, src1, dst1):
    f = pl.kernel(
        _edge_body,
        out_type=jax.ShapeDtypeStruct((N_E, 16), F32),
        mesh=_MESH,
        scratch_types=[
            pltpu.VMEM((EB2,), jnp.int32),
            pltpu.VMEM((EB2,), jnp.int32),
            pltpu.VMEM((EB2, 128), F32),
            pltpu.VMEM((EB2, 128), F32),
            pltpu.VMEM((EB2, 16), F32),
            pltpu.SemaphoreType.DMA,
            pltpu.SemaphoreType.DMA,
        ],
    )
    return f(hs, hd, src1, dst1)


BM = 2000
GRID = N_N // BM


def _norms(deg_ref):
    return lax.rsqrt(jnp.clip(deg_ref[...][:, :1], 1.0, None))


def _tc0_body(x_ref, wl_ref, bl_ref, w1_ref, dg_ref, *outs):
    t = jnp.dot(x_ref[...], wl_ref[...], preferred_element_type=F32) + bl_ref[...]
    p = jnp.dot(t, w1_ref[...], preferred_element_type=F32) * _norms(dg_ref)
    for i, o in enumerate(outs):
        o[...] = p[:, i * 128:(i + 1) * 128]


def _tc0(x, wlin, blin, w1, dego):
    return pl.pallas_call(
        _tc0_body,
        grid=(GRID,),
        in_specs=[
            pl.BlockSpec((BM, 256), lambda i: (i, 0)),
            pl.BlockSpec((256, 512), lambda i: (0, 0)),
            pl.BlockSpec((1, 512), lambda i: (0, 0)),
            pl.BlockSpec((512, 512), lambda i: (0, 0)),
            pl.BlockSpec((BM, 128), lambda i: (i, 0)),
        ],
        out_specs=[pl.BlockSpec((BM, 128), lambda i: (i, 0))] * 4,
        out_shape=[jax.ShapeDtypeStruct((N_N, 128), F32)] * 4,
    )(x, wlin, blin, w1, dego)


def _tc_mid(aggs, degi, dego, b, w_next, out_w, n_out, combine="concat"):
    if combine == "concat":
        in_w = sum(a.shape[1] for a in aggs)
    else:
        in_w = aggs[0].shape[1]
    f_out = w_next.shape[1]
    n_in = len(aggs)

    def body(*refs):
        agg_refs = refs[:n_in]
        di_ref, do_ref, b_ref, w_ref = refs[n_in:n_in + 4]
        outs = refs[n_in + 4:]
        if combine == "concat":
            agg = jnp.concatenate([r[...] for r in agg_refs], axis=1)
        else:
            agg = agg_refs[0][...]
            for r in agg_refs[1:]:
                agg = agg + r[...]
        y = jnp.maximum(agg * _norms(di_ref) + b_ref[...], 0.0)
        p = jnp.dot(y, w_ref[...], preferred_element_type=F32) * _norms(do_ref)
        for i, o in enumerate(outs):
            o[...] = p[:, i * out_w:(i + 1) * out_w]

    return pl.pallas_call(
        body,
        grid=(GRID,),
        in_specs=(
            [pl.BlockSpec((BM, a.shape[1]), lambda i: (i, 0)) for a in aggs]
            + [
                pl.BlockSpec((BM, 128), lambda i: (i, 0)),
                pl.BlockSpec((BM, 128), lambda i: (i, 0)),
                pl.BlockSpec((1, in_w), lambda i: (0, 0)),
                pl.BlockSpec((in_w, f_out), lambda i: (0, 0)),
            ]
        ),
        out_specs=[pl.BlockSpec((BM, out_w), lambda i: (i, 0))] * n_out,
        out_shape=[jax.ShapeDtypeStruct((N_N, out_w), F32)] * n_out,
    )(*aggs, degi, dego, b, w_next)


def _tc_last(aggs, degi, b4, w_top, w_bot, bcls):
    n_in = len(aggs)

    def body(*refs):
        agg_refs = refs[:n_in]
        di_ref, b_ref, wt_ref, wb_ref, bc_ref, hs_o, hd_o = refs[n_in:]
        agg = agg_refs[0][...]
        for r in agg_refs[1:]:
            agg = agg + r[...]
        y = jnp.maximum(agg * _norms(di_ref) + b_ref[...], 0.0)
        pad = jnp.zeros((BM, 112), F32)
        hs = jnp.dot(y, wt_ref[...], preferred_element_type=F32) + bc_ref[...]
        hd = jnp.dot(y, wb_ref[...], preferred_element_type=F32)
        hs_o[...] = jnp.concatenate([hs, pad], axis=1)
        hd_o[...] = jnp.concatenate([hd, pad], axis=1)

    return pl.pallas_call(
        body,
        grid=(GRID,),
        in_specs=(
            [pl.BlockSpec((BM, a.shape[1]), lambda i: (i, 0)) for a in aggs]
            + [
                pl.BlockSpec((BM, 128), lambda i: (i, 0)),
                pl.BlockSpec((1, 128), lambda i: (0, 0)),
                pl.BlockSpec((128, 16), lambda i: (0, 0)),
                pl.BlockSpec((128, 16), lambda i: (0, 0)),
                pl.BlockSpec((1, 16), lambda i: (0, 0)),
            ]
        ),
        out_specs=[pl.BlockSpec((BM, 128), lambda i: (i, 0))] * 2,
        out_shape=[jax.ShapeDtypeStruct((N_N, 128), F32)] * 2,
    )(*aggs, degi, b4, w_top, w_bot, bcls)


def _jax_deg(src, dst):
    dego = jnp.zeros((N_N,), F32).at[src].add(1.0)
    degi = jnp.zeros((N_N,), F32).at[dst].add(1.0)
    one = jnp.ones((1, 128), F32)
    return dego[:, None] * one, degi[:, None] * one


def _jax_spmm_chunks(src, dst, h_list):
    return [jnp.zeros((N_N, h.shape[1]), F32).at[dst].add(h[src]) for h in h_list]


def _jax_spmm_split(src, dst, h):
    half = N_E // 2
    p0 = jnp.zeros((N_N, 128), F32).at[dst[:half]].add(h[src[:half]])
    p1 = jnp.zeros((N_N, 128), F32).at[dst[half:]].add(h[src[half:]])
    return [p0, p1]


def _jax_edge(hs, hd, src, dst):
    return (hs[src] + hd[dst])[:, :16]


def kernel(node_features, edge_index, edge_features, W_lin, b_lin,
           W1, b1, W2, b2, W3, b3, W4, b4, W_cls, b_cls):
    del edge_features
    src = edge_index[0]
    dst = edge_index[1]
    z128 = jnp.zeros((CH, 128), F32)
    ones128 = jnp.ones((EB, 128), F32)

    dego, degi = _deg_call(src, dst, ones128, z128)

    p1 = _tc0(node_features, W_lin, b_lin.reshape(1, 512), W1, dego)
    a1 = _jax_spmm_chunks(src, dst, p1)
    p2 = _tc_mid(a1, degi, dego, b1.reshape(1, 512), W2, 128, 2)
    a2 = _jax_spmm_chunks(src, dst, p2)
    p3 = _tc_mid(a2, degi, dego, b2.reshape(1, 256), W3, 128, 1)
    a3 = _jax_spmm_split(src, dst, p3[0])
    p4 = _tc_mid(a3, degi, dego, b3.reshape(1, 128), W4, 128, 1, combine="sum")
    a4 = _jax_spmm_split(src, dst, p4[0])
    hs, hd = _tc_last(a4, degi, b4.reshape(1, 128), W_cls[:128], W_cls[128:],
                      b_cls.reshape(1, 16))
    return _jax_edge(hs, hd, src, dst)


# speedup vs baseline: 3.4014x; 3.3354x over previous
import functools

import jax
import jax.numpy as jnp
from jax import lax
from jax.experimental import pallas as pl
from jax.experimental.pallas import tpu as pltpu
from jax.experimental.pallas import tpu_sc as plsc

N_N = 10000
N_E = 160000
EB = 80
NB = 125
CH = 40
NCH = N_N // CH
F32 = jnp.float32

_MESH = plsc.VectorSubcoreMesh(core_axis_name="c", subcore_axis_name="s")


def _chunked(sid, fn):
    for z in range(16):
        ck = sid + 16 * z
        @pl.when(ck < NCH)
        def _(ck=ck):
            fn(ck)


def _deg_body(src_hbm, dst_hbm, ones_hbm, zeros_hbm, out0, out1,
              idx, ones_v, zer, wb, deg_sh):
    cid = lax.axis_index("c")
    sid = lax.axis_index("s")
    pltpu.sync_copy(ones_hbm, ones_v)
    pltpu.sync_copy(zeros_hbm, zer)
    for side in range(2):
        @pl.when(cid == side)
        def _(side=side):
            e_hbm = src_hbm if side == 0 else dst_hbm
            out_hbm = out0 if side == 0 else out1
            _chunked(sid, lambda ck: pltpu.sync_copy(
                zer, deg_sh.at[pl.ds(ck * CH, CH)]))
            plsc.subcore_barrier()

            def step(j, carry):
                base = sid * (NB * EB) + j * EB
                pltpu.sync_copy(e_hbm.at[pl.ds(base, EB)], idx)
                pltpu.sync_copy(ones_v, deg_sh.at[idx], add=True)
                return carry

            lax.fori_loop(0, NB, step, 0)
            plsc.subcore_barrier()

            def wback(ck):
                sl = pl.ds(ck * CH, CH)
                pltpu.sync_copy(deg_sh.at[sl], wb)
                pltpu.sync_copy(wb, out_hbm.at[sl])

            _chunked(sid, wback)


def _deg_call(src1, dst1, ones128, zeros128):
    f = pl.kernel(
        _deg_body,
        out_type=[jax.ShapeDtypeStruct((N_N, 128), F32)] * 2,
        mesh=_MESH,
        scratch_types=[
            pltpu.VMEM((EB,), jnp.int32),
            pltpu.VMEM((EB, 128), F32),
            pltpu.VMEM((CH, 128), F32),
            pltpu.VMEM((CH, 128), F32),
            pltpu.VMEM_SHARED((N_N, 128), F32),
        ],
    )
    return f(src1, dst1, ones128, zeros128)


def _make_spmm(n_chunks_per_core, width):
    C, W = n_chunks_per_core, width
    n_tab = 2 * C

    def body(src_hbm, dst_hbm, zeros_hbm, *rest):
        h_refs = rest[:n_tab]
        out_refs = rest[n_tab:2 * n_tab]
        sidx, didx, rows, zer, wb, agg, sem = rest[2 * n_tab:]
        cid = lax.axis_index("c")
        sid = lax.axis_index("s")
        pltpu.sync_copy(zeros_hbm, zer)
        for side in range(2):
            @pl.when(cid == side)
            def _(side=side):
                for k in range(C):
                    h_hbm = h_refs[side * C + k]
                    out_hbm = out_refs[side * C + k]
                    _chunked(sid, lambda ck: pltpu.sync_copy(
                        zer, agg.at[pl.ds(ck * CH, CH)]))
                    plsc.subcore_barrier()

                    def step(j, carry):
                        base = sid * (NB * EB) + j * EB
                        pltpu.sync_copy(src_hbm.at[pl.ds(base, EB)], sidx)
                        pltpu.sync_copy(dst_hbm.at[pl.ds(base, EB)], didx)
                        pltpu.async_copy(h_hbm.at[sidx], rows, sem).wait()
                        pltpu.sync_copy(rows, agg.at[didx], add=True)
                        return carry

                    lax.fori_loop(0, NB, step, 0)
                    plsc.subcore_barrier()

                    def wback(ck, out_hbm=out_hbm):
                        sl = pl.ds(ck * CH, CH)
                        pltpu.sync_copy(agg.at[sl], wb)
                        pltpu.sync_copy(wb, out_hbm.at[sl])

                    _chunked(sid, wback)
                    plsc.subcore_barrier()

    def call(src1, dst1, zeros_buf, h_list):
        f = pl.kernel(
            body,
            out_type=[jax.ShapeDtypeStruct((N_N, W), F32)] * n_tab,
            mesh=_MESH,
            scratch_types=[
                pltpu.VMEM((EB,), jnp.int32),
                pltpu.VMEM((EB,), jnp.int32),
                pltpu.VMEM((EB, W), F32),
                pltpu.VMEM((CH, W), F32),
                pltpu.VMEM((CH, W), F32),
                pltpu.VMEM_SHARED((N_N, W), F32),
                pltpu.SemaphoreType.DMA,
            ],
        )
        return f(src1, dst1, zeros_buf, *h_list)

    return call


_spmm_128x2 = _make_spmm(2, 128)
_spmm_128x1 = _make_spmm(1, 128)


EB3 = 40
NB3 = 125


def _spmm_split_body(src_hbm, dst_hbm, zeros_hbm, h_hbm, out0, out1,
                     sidx, didx, rows, zer, wb, agg, sem):
    cid = lax.axis_index("c")
    sid = lax.axis_index("s")
    pltpu.sync_copy(zeros_hbm, zer)
    for side in range(2):
        @pl.when(cid == side)
        def _(side=side):
            out_hbm = out0 if side == 0 else out1
            _chunked(sid, lambda ck: pltpu.sync_copy(
                zer, agg.at[pl.ds(ck * CH, CH)]))
            plsc.subcore_barrier()

            def step(j, carry):
                base = side * (N_E // 2) + sid * (NB3 * EB3) + j * EB3
                pltpu.sync_copy(src_hbm.at[pl.ds(base, EB3)], sidx)
                pltpu.sync_copy(dst_hbm.at[pl.ds(base, EB3)], didx)
                pltpu.async_copy(h_hbm.at[sidx], rows, sem).wait()
                pltpu.sync_copy(rows, agg.at[didx], add=True)
                return carry

            lax.fori_loop(0, NB3, step, 0)
            plsc.subcore_barrier()

            def wback(ck, out_hbm=out_hbm):
                sl = pl.ds(ck * CH, CH)
                pltpu.sync_copy(agg.at[sl], wb)
                pltpu.sync_copy(wb, out_hbm.at[sl])

            _chunked(sid, wback)


def _spmm_split(src1, dst1, zeros_buf, h):
    f = pl.kernel(
        _spmm_split_body,
        out_type=[jax.ShapeDtypeStruct((N_N, 128), F32)] * 2,
        mesh=_MESH,
        scratch_types=[
            pltpu.VMEM((EB3,), jnp.int32),
            pltpu.VMEM((EB3,), jnp.int32),
            pltpu.VMEM((EB3, 128), F32),
            pltpu.VMEM((CH, 128), F32),
            pltpu.VMEM((CH, 128), F32),
            pltpu.VMEM_SHARED((N_N, 128), F32),
            pltpu.SemaphoreType.DMA,
        ],
    )
    return f(src1, dst1, zeros_buf, h)


EB2 = 40
NB2 = 125


def _edge_body(hs_hbm, hd_hbm, src_hbm, dst_hbm, out_hbm,
               sidx, didx, a_v, b_v, o_v, sem_a, sem_b):
    cid = lax.axis_index("c")
    sid = lax.axis_index("s")
    wid = sid * 2 + cid

    def step(j, carry):
        base = wid * (NB2 * EB2) + j * EB2
        pltpu.sync_copy(src_hbm.at[pl.ds(base, EB2)], sidx)
        pltpu.sync_copy(dst_hbm.at[pl.ds(base, EB2)], didx)
        da = pltpu.async_copy(hs_hbm.at[sidx], a_v, sem_a)
        db = pltpu.async_copy(hd_hbm.at[didx], b_v, sem_b)
        da.wait()
        db.wait()
        for r in range(EB2):
            o_v[r, :] = a_v[r, pl.ds(0, 16)] + b_v[r, pl.ds(0, 16)]
        pltpu.sync_copy(o_v, out_hbm.at[pl.ds(wid * 5000 + j * EB2, EB2)])
        return carry

    lax.fori_loop(0, NB2, step, 0)


def _edge_call(hs, hd, src1, dst1):
    f = pl.kernel(
        _edge_body,
        out_type=jax.ShapeDtypeStruct((N_E, 16), F32),
        mesh=_MESH,
        scratch_types=[
            pltpu.VMEM((EB2,), jnp.int32),
            pltpu.VMEM((EB2,), jnp.int32),
            pltpu.VMEM((EB2, 128), F32),
            pltpu.VMEM((EB2, 128), F32),
            pltpu.VMEM((EB2, 16), F32),
            pltpu.SemaphoreType.DMA,
            pltpu.SemaphoreType.DMA,
        ],
    )
    return f(hs, hd, src1, dst1)


BM = 2000
GRID = N_N // BM


def _norms(deg_ref):
    return lax.rsqrt(jnp.clip(deg_ref[...][:, :1], 1.0, None))


def _tc0_body(x_ref, wl_ref, bl_ref, w1_ref, dg_ref, *outs):
    t = jnp.dot(x_ref[...], wl_ref[...], preferred_element_type=F32) + bl_ref[...]
    p = jnp.dot(t, w1_ref[...], preferred_element_type=F32) * _norms(dg_ref)
    for i, o in enumerate(outs):
        o[...] = p[:, i * 128:(i + 1) * 128]


def _tc0(x, wlin, blin, w1, dego):
    return pl.pallas_call(
        _tc0_body,
        grid=(GRID,),
        in_specs=[
            pl.BlockSpec((BM, 256), lambda i: (i, 0)),
            pl.BlockSpec((256, 512), lambda i: (0, 0)),
            pl.BlockSpec((1, 512), lambda i: (0, 0)),
            pl.BlockSpec((512, 512), lambda i: (0, 0)),
            pl.BlockSpec((BM, 128), lambda i: (i, 0)),
        ],
        out_specs=[pl.BlockSpec((BM, 128), lambda i: (i, 0))] * 4,
        out_shape=[jax.ShapeDtypeStruct((N_N, 128), F32)] * 4,
    )(x, wlin, blin, w1, dego)


def _tc_mid(aggs, degi, dego, b, w_next, out_w, n_out, combine="concat"):
    if combine == "concat":
        in_w = sum(a.shape[1] for a in aggs)
    else:
        in_w = aggs[0].shape[1]
    f_out = w_next.shape[1]
    n_in = len(aggs)

    def body(*refs):
        agg_refs = refs[:n_in]
        di_ref, do_ref, b_ref, w_ref = refs[n_in:n_in + 4]
        outs = refs[n_in + 4:]
        if combine == "concat":
            agg = jnp.concatenate([r[...] for r in agg_refs], axis=1)
        else:
            agg = agg_refs[0][...]
            for r in agg_refs[1:]:
                agg = agg + r[...]
        y = jnp.maximum(agg * _norms(di_ref) + b_ref[...], 0.0)
        p = jnp.dot(y, w_ref[...], preferred_element_type=F32) * _norms(do_ref)
        for i, o in enumerate(outs):
            o[...] = p[:, i * out_w:(i + 1) * out_w]

    return pl.pallas_call(
        body,
        grid=(GRID,),
        in_specs=(
            [pl.BlockSpec((BM, a.shape[1]), lambda i: (i, 0)) for a in aggs]
            + [
                pl.BlockSpec((BM, 128), lambda i: (i, 0)),
                pl.BlockSpec((BM, 128), lambda i: (i, 0)),
                pl.BlockSpec((1, in_w), lambda i: (0, 0)),
                pl.BlockSpec((in_w, f_out), lambda i: (0, 0)),
            ]
        ),
        out_specs=[pl.BlockSpec((BM, out_w), lambda i: (i, 0))] * n_out,
        out_shape=[jax.ShapeDtypeStruct((N_N, out_w), F32)] * n_out,
    )(*aggs, degi, dego, b, w_next)


def _tc_last(aggs, degi, b4, w_top, w_bot, bcls):
    n_in = len(aggs)

    def body(*refs):
        agg_refs = refs[:n_in]
        di_ref, b_ref, wt_ref, wb_ref, bc_ref, hs_o, hd_o = refs[n_in:]
        agg = agg_refs[0][...]
        for r in agg_refs[1:]:
            agg = agg + r[...]
        y = jnp.maximum(agg * _norms(di_ref) + b_ref[...], 0.0)
        pad = jnp.zeros((BM, 112), F32)
        hs = jnp.dot(y, wt_ref[...], preferred_element_type=F32) + bc_ref[...]
        hd = jnp.dot(y, wb_ref[...], preferred_element_type=F32)
        hs_o[...] = jnp.concatenate([hs, pad], axis=1)
        hd_o[...] = jnp.concatenate([hd, pad], axis=1)

    return pl.pallas_call(
        body,
        grid=(GRID,),
        in_specs=(
            [pl.BlockSpec((BM, a.shape[1]), lambda i: (i, 0)) for a in aggs]
            + [
                pl.BlockSpec((BM, 128), lambda i: (i, 0)),
                pl.BlockSpec((1, 128), lambda i: (0, 0)),
                pl.BlockSpec((128, 16), lambda i: (0, 0)),
                pl.BlockSpec((128, 16), lambda i: (0, 0)),
                pl.BlockSpec((1, 16), lambda i: (0, 0)),
            ]
        ),
        out_specs=[pl.BlockSpec((BM, 128), lambda i: (i, 0))] * 2,
        out_shape=[jax.ShapeDtypeStruct((N_N, 128), F32)] * 2,
    )(*aggs, degi, b4, w_top, w_bot, bcls)


def _jax_deg(src, dst):
    dego = jnp.zeros((N_N,), F32).at[src].add(1.0)
    degi = jnp.zeros((N_N,), F32).at[dst].add(1.0)
    one = jnp.ones((1, 128), F32)
    return dego[:, None] * one, degi[:, None] * one


def _jax_spmm_chunks(src, dst, h_list):
    return [jnp.zeros((N_N, h.shape[1]), F32).at[dst].add(h[src]) for h in h_list]


def _jax_spmm_split(src, dst, h):
    half = N_E // 2
    p0 = jnp.zeros((N_N, 128), F32).at[dst[:half]].add(h[src[:half]])
    p1 = jnp.zeros((N_N, 128), F32).at[dst[half:]].add(h[src[half:]])
    return [p0, p1]


def _jax_edge(hs, hd, src, dst):
    return (hs[src] + hd[dst])[:, :16]


def kernel(node_features, edge_index, edge_features, W_lin, b_lin,
           W1, b1, W2, b2, W3, b3, W4, b4, W_cls, b_cls):
    del edge_features
    src = edge_index[0]
    dst = edge_index[1]
    z128 = jnp.zeros((CH, 128), F32)
    ones128 = jnp.ones((EB, 128), F32)

    dego, degi = _deg_call(src, dst, ones128, z128)

    p1 = _tc0(node_features, W_lin, b_lin.reshape(1, 512), W1, dego)
    a1 = _spmm_128x2(src, dst, z128, p1)
    p2 = _tc_mid(a1, degi, dego, b1.reshape(1, 512), W2, 128, 2)
    a2 = _spmm_128x1(src, dst, z128, p2)
    p3 = _tc_mid(a2, degi, dego, b2.reshape(1, 256), W3, 128, 1)
    a3 = _spmm_split(src, dst, z128, p3[0])
    p4 = _tc_mid(a3, degi, dego, b3.reshape(1, 128), W4, 128, 1, combine="sum")
    a4 = _spmm_split(src, dst, z128, p4[0])
    hs, hd = _tc_last(a4, degi, b4.reshape(1, 128), W_cls[:128], W_cls[128:],
                      b_cls.reshape(1, 16))
    return _edge_call(hs, hd, src, dst)


# speedup vs baseline: 6.1682x; 1.8134x over previous
import functools

import jax
import jax.numpy as jnp
from jax import lax
from jax.experimental import pallas as pl
from jax.experimental.pallas import tpu as pltpu
from jax.experimental.pallas import tpu_sc as plsc

N_N = 10000
N_E = 160000
EB = 80
NB = 125
CH = 40
NCH = N_N // CH
F32 = jnp.float32

_MESH = plsc.VectorSubcoreMesh(core_axis_name="c", subcore_axis_name="s")


def _chunked(sid, fn):
    for z in range(16):
        ck = sid + 16 * z
        @pl.when(ck < NCH)
        def _(ck=ck):
            fn(ck)


PIPE = 3


def _deg_body(src_hbm, dst_hbm, ones_hbm, zeros_hbm, out0, out1,
              idx, ones_v, zer, deg_sh, *sems):
    sem_i = sems[:PIPE]
    sem_s = sems[PIPE:]
    cid = lax.axis_index("c")
    sid = lax.axis_index("s")
    pltpu.sync_copy(ones_hbm, ones_v)
    pltpu.sync_copy(zeros_hbm, zer)
    for side in range(2):
        @pl.when(cid == side)
        def _(side=side):
            e_hbm = src_hbm if side == 0 else dst_hbm
            out_hbm = out0 if side == 0 else out1
            _chunked(sid, lambda ck: pltpu.sync_copy(
                zer, deg_sh.at[pl.ds(ck * CH, CH)]))
            plsc.subcore_barrier()

            def step(k, carry):
                for t in range(PIPE):
                    j = k * PIPE + t
                    @pl.when(j < NB)
                    def _(j=j, t=t):
                        base = sid * (NB * EB) + j * EB
                        pltpu.make_async_copy(
                            e_hbm.at[pl.ds(base, EB)], idx.at[t], sem_i[t]).start()
                for t in range(PIPE):
                    j = k * PIPE + t
                    @pl.when(j < NB)
                    def _(j=j, t=t):
                        base = sid * (NB * EB) + j * EB
                        pltpu.make_async_copy(
                            e_hbm.at[pl.ds(base, EB)], idx.at[t], sem_i[t]).wait()
                        pltpu.make_async_copy(
                            ones_v, deg_sh.at[idx.at[t]], sem_s[t]).start(add=True)
                for t in range(PIPE):
                    j = k * PIPE + t
                    @pl.when(j < NB)
                    def _(j=j, t=t):
                        pltpu.make_async_copy(
                            ones_v, deg_sh.at[idx.at[t]], sem_s[t]).wait()
                return carry

            lax.fori_loop(0, (NB + PIPE - 1) // PIPE, step, 0)
            plsc.subcore_barrier()

            def wback(ck):
                sl = pl.ds(ck * CH, CH)
                pltpu.sync_copy(deg_sh.at[sl], zer)
                pltpu.sync_copy(zer, out_hbm.at[sl])

            _chunked(sid, wback)


def _deg_call(src1, dst1, ones128, zeros128):
    f = pl.kernel(
        _deg_body,
        out_type=[jax.ShapeDtypeStruct((N_N, 128), F32)] * 2,
        mesh=_MESH,
        scratch_types=[
            pltpu.VMEM((PIPE, EB), jnp.int32),
            pltpu.VMEM((EB, 128), F32),
            pltpu.VMEM((CH, 128), F32),
            pltpu.VMEM_SHARED((N_N, 128), F32),
        ] + [pltpu.SemaphoreType.DMA] * (2 * PIPE),
    )
    return f(src1, dst1, ones128, zeros128)


def _make_spmm(n_chunks_per_core, width):
    C, W = n_chunks_per_core, width
    n_tab = 2 * C

    def body(src_hbm, dst_hbm, zeros_hbm, *rest):
        h_refs = rest[:n_tab]
        out_refs = rest[n_tab:2 * n_tab]
        sidx, didx, rows, zer, agg = rest[2 * n_tab:2 * n_tab + 5]
        sems = rest[2 * n_tab + 5:]
        sem_is, sem_id = sems[:PIPE], sems[PIPE:2 * PIPE]
        sem_g, sem_s = sems[2 * PIPE:3 * PIPE], sems[3 * PIPE:]
        cid = lax.axis_index("c")
        sid = lax.axis_index("s")
        pltpu.sync_copy(zeros_hbm, zer)
        for side in range(2):
            @pl.when(cid == side)
            def _(side=side):
                for k in range(C):
                    h_hbm = h_refs[side * C + k]
                    out_hbm = out_refs[side * C + k]
                    _chunked(sid, lambda ck: pltpu.sync_copy(
                        zer, agg.at[pl.ds(ck * CH, CH)]))
                    plsc.subcore_barrier()

                    def step(kk, carry, h_hbm=h_hbm):
                        for t in range(PIPE):
                            j = kk * PIPE + t
                            @pl.when(j < NB)
                            def _(j=j, t=t):
                                base = sid * (NB * EB) + j * EB
                                pltpu.make_async_copy(
                                    src_hbm.at[pl.ds(base, EB)], sidx.at[t],
                                    sem_is[t]).start()
                                pltpu.make_async_copy(
                                    dst_hbm.at[pl.ds(base, EB)], didx.at[t],
                                    sem_id[t]).start()
                        for t in range(PIPE):
                            j = kk * PIPE + t
                            @pl.when(j < NB)
                            def _(j=j, t=t):
                                base = sid * (NB * EB) + j * EB
                                pltpu.make_async_copy(
                                    src_hbm.at[pl.ds(base, EB)], sidx.at[t],
                                    sem_is[t]).wait()
                                pltpu.make_async_copy(
                                    h_hbm.at[sidx.at[t]], rows.at[t],
                                    sem_g[t]).start()
                        for t in range(PIPE):
                            j = kk * PIPE + t
                            @pl.when(j < NB)
                            def _(j=j, t=t):
                                base = sid * (NB * EB) + j * EB
                                pltpu.make_async_copy(
                                    dst_hbm.at[pl.ds(base, EB)], didx.at[t],
                                    sem_id[t]).wait()
                                pltpu.make_async_copy(
                                    h_hbm.at[sidx.at[t]], rows.at[t],
                                    sem_g[t]).wait()
                                pltpu.make_async_copy(
                                    rows.at[t], agg.at[didx.at[t]],
                                    sem_s[t]).start(add=True)
                        for t in range(PIPE):
                            j = kk * PIPE + t
                            @pl.when(j < NB)
                            def _(j=j, t=t):
                                pltpu.make_async_copy(
                                    rows.at[t], agg.at[didx.at[t]],
                                    sem_s[t]).wait()
                        return carry

                    lax.fori_loop(0, (NB + PIPE - 1) // PIPE, step, 0)
                    plsc.subcore_barrier()

                    def wback(ck, out_hbm=out_hbm):
                        sl = pl.ds(ck * CH, CH)
                        pltpu.sync_copy(agg.at[sl], zer)
                        pltpu.sync_copy(zer, out_hbm.at[sl])

                    _chunked(sid, wback)
                    plsc.subcore_barrier()
                    if k + 1 < C:
                        pltpu.sync_copy(zeros_hbm, zer)

    def call(src1, dst1, zeros_buf, h_list):
        f = pl.kernel(
            body,
            out_type=[jax.ShapeDtypeStruct((N_N, W), F32)] * n_tab,
            mesh=_MESH,
            scratch_types=[
                pltpu.VMEM((PIPE, EB), jnp.int32),
                pltpu.VMEM((PIPE, EB), jnp.int32),
                pltpu.VMEM((PIPE, EB, W), F32),
                pltpu.VMEM((CH, W), F32),
                pltpu.VMEM_SHARED((N_N, W), F32),
            ] + [pltpu.SemaphoreType.DMA] * (4 * PIPE),
        )
        return f(src1, dst1, zeros_buf, *h_list)

    return call


_spmm_128x2 = _make_spmm(2, 128)
_spmm_128x1 = _make_spmm(1, 128)


EB3 = 40
NB3 = 125


def _spmm_split_body(src_hbm, dst_hbm, zeros_hbm, h_hbm, out0, out1,
                     sidx, didx, rows, zer, agg, *sems):
    sem_is, sem_id = sems[:PIPE], sems[PIPE:2 * PIPE]
    sem_g, sem_s = sems[2 * PIPE:3 * PIPE], sems[3 * PIPE:]
    cid = lax.axis_index("c")
    sid = lax.axis_index("s")
    pltpu.sync_copy(zeros_hbm, zer)
    for side in range(2):
        @pl.when(cid == side)
        def _(side=side):
            out_hbm = out0 if side == 0 else out1
            _chunked(sid, lambda ck: pltpu.sync_copy(
                zer, agg.at[pl.ds(ck * CH, CH)]))
            plsc.subcore_barrier()

            def step(kk, carry):
                for t in range(PIPE):
                    j = kk * PIPE + t
                    @pl.when(j < NB3)
                    def _(j=j, t=t):
                        base = side * (N_E // 2) + sid * (NB3 * EB3) + j * EB3
                        pltpu.make_async_copy(
                            src_hbm.at[pl.ds(base, EB3)], sidx.at[t],
                            sem_is[t]).start()
                        pltpu.make_async_copy(
                            dst_hbm.at[pl.ds(base, EB3)], didx.at[t],
                            sem_id[t]).start()
                for t in range(PIPE):
                    j = kk * PIPE + t
                    @pl.when(j < NB3)
                    def _(j=j, t=t):
                        base = side * (N_E // 2) + sid * (NB3 * EB3) + j * EB3
                        pltpu.make_async_copy(
                            src_hbm.at[pl.ds(base, EB3)], sidx.at[t],
                            sem_is[t]).wait()
                        pltpu.make_async_copy(
                            h_hbm.at[sidx.at[t]], rows.at[t], sem_g[t]).start()
                for t in range(PIPE):
                    j = kk * PIPE + t
                    @pl.when(j < NB3)
                    def _(j=j, t=t):
                        base = side * (N_E // 2) + sid * (NB3 * EB3) + j * EB3
                        pltpu.make_async_copy(
                            dst_hbm.at[pl.ds(base, EB3)], didx.at[t],
                            sem_id[t]).wait()
                        pltpu.make_async_copy(
                            h_hbm.at[sidx.at[t]], rows.at[t], sem_g[t]).wait()
                        pltpu.make_async_copy(
                            rows.at[t], agg.at[didx.at[t]],
                            sem_s[t]).start(add=True)
                for t in range(PIPE):
                    j = kk * PIPE + t
                    @pl.when(j < NB3)
                    def _(j=j, t=t):
                        pltpu.make_async_copy(
                            rows.at[t], agg.at[didx.at[t]], sem_s[t]).wait()
                return carry

            lax.fori_loop(0, (NB3 + PIPE - 1) // PIPE, step, 0)
            plsc.subcore_barrier()

            def wback(ck, out_hbm=out_hbm):
                sl = pl.ds(ck * CH, CH)
                pltpu.sync_copy(agg.at[sl], zer)
                pltpu.sync_copy(zer, out_hbm.at[sl])

            _chunked(sid, wback)


def _spmm_split(src1, dst1, zeros_buf, h):
    f = pl.kernel(
        _spmm_split_body,
        out_type=[jax.ShapeDtypeStruct((N_N, 128), F32)] * 2,
        mesh=_MESH,
        scratch_types=[
            pltpu.VMEM((PIPE, EB3), jnp.int32),
            pltpu.VMEM((PIPE, EB3), jnp.int32),
            pltpu.VMEM((PIPE, EB3, 128), F32),
            pltpu.VMEM((CH, 128), F32),
            pltpu.VMEM_SHARED((N_N, 128), F32),
        ] + [pltpu.SemaphoreType.DMA] * (4 * PIPE),
    )
    return f(src1, dst1, zeros_buf, h)


EB2 = 40
NB2 = 125


EP = 2


def _edge_body(hs_hbm, hd_hbm, src_hbm, dst_hbm, out_hbm,
               sidx, didx, a_v, b_v, o_v, *sems):
    sem_is, sem_id = sems[:EP], sems[EP:2 * EP]
    sem_a, sem_b, sem_o = sems[2 * EP:3 * EP], sems[3 * EP:4 * EP], sems[4 * EP:]
    cid = lax.axis_index("c")
    sid = lax.axis_index("s")
    wid = sid * 2 + cid

    def step(kk, carry):
        for t in range(EP):
            j = kk * EP + t
            @pl.when(j < NB2)
            def _(j=j, t=t):
                base = wid * (NB2 * EB2) + j * EB2
                pltpu.make_async_copy(
                    src_hbm.at[pl.ds(base, EB2)], sidx.at[t], sem_is[t]).start()
                pltpu.make_async_copy(
                    dst_hbm.at[pl.ds(base, EB2)], didx.at[t], sem_id[t]).start()
        for t in range(EP):
            j = kk * EP + t
            @pl.when(j < NB2)
            def _(j=j, t=t):
                base = wid * (NB2 * EB2) + j * EB2
                pltpu.make_async_copy(
                    src_hbm.at[pl.ds(base, EB2)], sidx.at[t], sem_is[t]).wait()
                pltpu.make_async_copy(
                    dst_hbm.at[pl.ds(base, EB2)], didx.at[t], sem_id[t]).wait()
                pltpu.make_async_copy(
                    hs_hbm.at[sidx.at[t]], a_v.at[t], sem_a[t]).start()
                pltpu.make_async_copy(
                    hd_hbm.at[didx.at[t]], b_v.at[t], sem_b[t]).start()
        for t in range(EP):
            j = kk * EP + t
            @pl.when(j < NB2)
            def _(j=j, t=t):
                pltpu.make_async_copy(
                    hs_hbm.at[sidx.at[t]], a_v.at[t], sem_a[t]).wait()
                pltpu.make_async_copy(
                    hd_hbm.at[didx.at[t]], b_v.at[t], sem_b[t]).wait()
                for r in range(EB2):
                    o_v[t, r, :] = (a_v[t, r, pl.ds(0, 16)]
                                    + b_v[t, r, pl.ds(0, 16)])
                pltpu.make_async_copy(
                    o_v.at[t],
                    out_hbm.at[pl.ds(wid * (NB2 * EB2) + j * EB2, EB2)],
                    sem_o[t]).start()
        for t in range(EP):
            j = kk * EP + t
            @pl.when(j < NB2)
            def _(j=j, t=t):
                pltpu.make_async_copy(
                    o_v.at[t],
                    out_hbm.at[pl.ds(wid * (NB2 * EB2) + j * EB2, EB2)],
                    sem_o[t]).wait()
        return carry

    lax.fori_loop(0, (NB2 + EP - 1) // EP, step, 0)


def _edge_call(hs, hd, src1, dst1):
    f = pl.kernel(
        _edge_body,
        out_type=jax.ShapeDtypeStruct((N_E, 16), F32),
        mesh=_MESH,
        scratch_types=[
            pltpu.VMEM((EP, EB2), jnp.int32),
            pltpu.VMEM((EP, EB2), jnp.int32),
            pltpu.VMEM((EP, EB2, 128), F32),
            pltpu.VMEM((EP, EB2, 128), F32),
            pltpu.VMEM((EP, EB2, 16), F32),
        ] + [pltpu.SemaphoreType.DMA] * (5 * EP),
    )
    return f(hs, hd, src1, dst1)


BM = 2000
GRID = N_N // BM


def _norms(deg_ref):
    return lax.rsqrt(jnp.clip(deg_ref[...][:, :1], 1.0, None))


def _tc0_body(x_ref, wl_ref, bl_ref, w1_ref, dg_ref, *outs):
    t = jnp.dot(x_ref[...], wl_ref[...], preferred_element_type=F32) + bl_ref[...]
    p = jnp.dot(t, w1_ref[...], preferred_element_type=F32) * _norms(dg_ref)
    for i, o in enumerate(outs):
        o[...] = p[:, i * 128:(i + 1) * 128]


def _tc0(x, wlin, blin, w1, dego):
    return pl.pallas_call(
        _tc0_body,
        grid=(GRID,),
        in_specs=[
            pl.BlockSpec((BM, 256), lambda i: (i, 0)),
            pl.BlockSpec((256, 512), lambda i: (0, 0)),
            pl.BlockSpec((1, 512), lambda i: (0, 0)),
            pl.BlockSpec((512, 512), lambda i: (0, 0)),
            pl.BlockSpec((BM, 128), lambda i: (i, 0)),
        ],
        out_specs=[pl.BlockSpec((BM, 128), lambda i: (i, 0))] * 4,
        out_shape=[jax.ShapeDtypeStruct((N_N, 128), F32)] * 4,
    )(x, wlin, blin, w1, dego)


def _tc_mid(aggs, degi, dego, b, w_next, out_w, n_out, combine="concat"):
    if combine == "concat":
        in_w = sum(a.shape[1] for a in aggs)
    else:
        in_w = aggs[0].shape[1]
    f_out = w_next.shape[1]
    n_in = len(aggs)

    def body(*refs):
        agg_refs = refs[:n_in]
        di_ref, do_ref, b_ref, w_ref = refs[n_in:n_in + 4]
        outs = refs[n_in + 4:]
        if combine == "concat":
            agg = jnp.concatenate([r[...] for r in agg_refs], axis=1)
        else:
            agg = agg_refs[0][...]
            for r in agg_refs[1:]:
                agg = agg + r[...]
        y = jnp.maximum(agg * _norms(di_ref) + b_ref[...], 0.0)
        p = jnp.dot(y, w_ref[...], preferred_element_type=F32) * _norms(do_ref)
        for i, o in enumerate(outs):
            o[...] = p[:, i * out_w:(i + 1) * out_w]

    return pl.pallas_call(
        body,
        grid=(GRID,),
        in_specs=(
            [pl.BlockSpec((BM, a.shape[1]), lambda i: (i, 0)) for a in aggs]
            + [
                pl.BlockSpec((BM, 128), lambda i: (i, 0)),
                pl.BlockSpec((BM, 128), lambda i: (i, 0)),
                pl.BlockSpec((1, in_w), lambda i: (0, 0)),
                pl.BlockSpec((in_w, f_out), lambda i: (0, 0)),
            ]
        ),
        out_specs=[pl.BlockSpec((BM, out_w), lambda i: (i, 0))] * n_out,
        out_shape=[jax.ShapeDtypeStruct((N_N, out_w), F32)] * n_out,
    )(*aggs, degi, dego, b, w_next)


def _tc_last(aggs, degi, b4, w_top, w_bot, bcls):
    n_in = len(aggs)

    def body(*refs):
        agg_refs = refs[:n_in]
        di_ref, b_ref, wt_ref, wb_ref, bc_ref, hs_o, hd_o = refs[n_in:]
        agg = agg_refs[0][...]
        for r in agg_refs[1:]:
            agg = agg + r[...]
        y = jnp.maximum(agg * _norms(di_ref) + b_ref[...], 0.0)
        pad = jnp.zeros((BM, 112), F32)
        hs = jnp.dot(y, wt_ref[...], preferred_element_type=F32) + bc_ref[...]
        hd = jnp.dot(y, wb_ref[...], preferred_element_type=F32)
        hs_o[...] = jnp.concatenate([hs, pad], axis=1)
        hd_o[...] = jnp.concatenate([hd, pad], axis=1)

    return pl.pallas_call(
        body,
        grid=(GRID,),
        in_specs=(
            [pl.BlockSpec((BM, a.shape[1]), lambda i: (i, 0)) for a in aggs]
            + [
                pl.BlockSpec((BM, 128), lambda i: (i, 0)),
                pl.BlockSpec((1, 128), lambda i: (0, 0)),
                pl.BlockSpec((128, 16), lambda i: (0, 0)),
                pl.BlockSpec((128, 16), lambda i: (0, 0)),
                pl.BlockSpec((1, 16), lambda i: (0, 0)),
            ]
        ),
        out_specs=[pl.BlockSpec((BM, 128), lambda i: (i, 0))] * 2,
        out_shape=[jax.ShapeDtypeStruct((N_N, 128), F32)] * 2,
    )(*aggs, degi, b4, w_top, w_bot, bcls)


def _jax_deg(src, dst):
    dego = jnp.zeros((N_N,), F32).at[src].add(1.0)
    degi = jnp.zeros((N_N,), F32).at[dst].add(1.0)
    one = jnp.ones((1, 128), F32)
    return dego[:, None] * one, degi[:, None] * one


def _jax_spmm_chunks(src, dst, h_list):
    return [jnp.zeros((N_N, h.shape[1]), F32).at[dst].add(h[src]) for h in h_list]


def _jax_spmm_split(src, dst, h):
    half = N_E // 2
    p0 = jnp.zeros((N_N, 128), F32).at[dst[:half]].add(h[src[:half]])
    p1 = jnp.zeros((N_N, 128), F32).at[dst[half:]].add(h[src[half:]])
    return [p0, p1]


def _jax_edge(hs, hd, src, dst):
    return (hs[src] + hd[dst])[:, :16]


def kernel(node_features, edge_index, edge_features, W_lin, b_lin,
           W1, b1, W2, b2, W3, b3, W4, b4, W_cls, b_cls):
    del edge_features
    src = edge_index[0]
    dst = edge_index[1]
    z128 = jnp.zeros((CH, 128), F32)
    ones128 = jnp.ones((EB, 128), F32)

    dego, degi = _deg_call(src, dst, ones128, z128)

    p1 = _tc0(node_features, W_lin, b_lin.reshape(1, 512), W1, dego)
    a1 = _spmm_128x2(src, dst, z128, p1)
    p2 = _tc_mid(a1, degi, dego, b1.reshape(1, 512), W2, 128, 2)
    a2 = _spmm_128x1(src, dst, z128, p2)
    p3 = _tc_mid(a2, degi, dego, b2.reshape(1, 256), W3, 128, 1)
    a3 = _spmm_split(src, dst, z128, p3[0])
    p4 = _tc_mid(a3, degi, dego, b3.reshape(1, 128), W4, 128, 1, combine="sum")
    a4 = _spmm_split(src, dst, z128, p4[0])
    hs, hd = _tc_last(a4, degi, b4.reshape(1, 128), W_cls[:128], W_cls[128:],
                      b_cls.reshape(1, 16))
    return _edge_call(hs, hd, src, dst)


# speedup vs baseline: 6.1713x; 1.0005x over previous
import functools

import jax
import jax.numpy as jnp
from jax import lax
from jax.experimental import pallas as pl
from jax.experimental.pallas import tpu as pltpu
from jax.experimental.pallas import tpu_sc as plsc

N_N = 10000
N_E = 160000
EB = 80
NB = 125
CH = 40
NCH = N_N // CH
F32 = jnp.float32

_MESH = plsc.VectorSubcoreMesh(core_axis_name="c", subcore_axis_name="s")


def _chunked(sid, fn):
    for z in range(16):
        ck = sid + 16 * z
        @pl.when(ck < NCH)
        def _(ck=ck):
            fn(ck)


PIPE = 3
DPIPE = 3
SPIPE = 3
EP = 2


def _deg_body(src_hbm, dst_hbm, ones_hbm, zeros_hbm, out0, out1,
              idx, ones_v, zer, deg_sh, *sems):
    sem_i = sems[:DPIPE]
    sem_s = sems[DPIPE:]
    cid = lax.axis_index("c")
    sid = lax.axis_index("s")
    pltpu.sync_copy(ones_hbm, ones_v)
    pltpu.sync_copy(zeros_hbm, zer)
    for side in range(2):
        @pl.when(cid == side)
        def _(side=side):
            e_hbm = src_hbm if side == 0 else dst_hbm
            out_hbm = out0 if side == 0 else out1
            _chunked(sid, lambda ck: pltpu.sync_copy(
                zer, deg_sh.at[pl.ds(ck * CH, CH)]))
            plsc.subcore_barrier()

            def step(k, carry):
                for t in range(DPIPE):
                    j = k * DPIPE + t
                    @pl.when(j < NB)
                    def _(j=j, t=t):
                        base = sid * (NB * EB) + j * EB
                        pltpu.make_async_copy(
                            e_hbm.at[pl.ds(base, EB)], idx.at[t], sem_i[t]).start()
                for t in range(DPIPE):
                    j = k * DPIPE + t
                    @pl.when(j < NB)
                    def _(j=j, t=t):
                        base = sid * (NB * EB) + j * EB
                        pltpu.make_async_copy(
                            e_hbm.at[pl.ds(base, EB)], idx.at[t], sem_i[t]).wait()
                        pltpu.make_async_copy(
                            ones_v, deg_sh.at[idx.at[t]], sem_s[t]).start(add=True)
                for t in range(DPIPE):
                    j = k * DPIPE + t
                    @pl.when(j < NB)
                    def _(j=j, t=t):
                        pltpu.make_async_copy(
                            ones_v, deg_sh.at[idx.at[t]], sem_s[t]).wait()
                return carry

            lax.fori_loop(0, (NB + DPIPE - 1) // DPIPE, step, 0)
            plsc.subcore_barrier()

            def wback(ck):
                sl = pl.ds(ck * CH, CH)
                pltpu.sync_copy(deg_sh.at[sl], zer)
                pltpu.sync_copy(zer, out_hbm.at[sl])

            _chunked(sid, wback)


def _deg_call(src1, dst1, ones128, zeros128):
    f = pl.kernel(
        _deg_body,
        out_type=[jax.ShapeDtypeStruct((N_N, 128), F32)] * 2,
        mesh=_MESH,
        scratch_types=[
            pltpu.VMEM((DPIPE, EB), jnp.int32),
            pltpu.VMEM((EB, 128), F32),
            pltpu.VMEM((CH, 128), F32),
            pltpu.VMEM_SHARED((N_N, 128), F32),
        ] + [pltpu.SemaphoreType.DMA] * (2 * DPIPE),
    )
    return f(src1, dst1, ones128, zeros128)


def _make_spmm(n_chunks_per_core, width):
    C, W = n_chunks_per_core, width
    n_tab = 2 * C

    def body(src_hbm, dst_hbm, zeros_hbm, *rest):
        h_refs = rest[:n_tab]
        out_refs = rest[n_tab:2 * n_tab]
        sidx, didx, rows, zer, agg = rest[2 * n_tab:2 * n_tab + 5]
        sems = rest[2 * n_tab + 5:]
        sem_is, sem_id = sems[:PIPE], sems[PIPE:2 * PIPE]
        sem_g, sem_s = sems[2 * PIPE:3 * PIPE], sems[3 * PIPE:]
        cid = lax.axis_index("c")
        sid = lax.axis_index("s")
        pltpu.sync_copy(zeros_hbm, zer)
        for side in range(2):
            @pl.when(cid == side)
            def _(side=side):
                for k in range(C):
                    h_hbm = h_refs[side * C + k]
                    out_hbm = out_refs[side * C + k]
                    _chunked(sid, lambda ck: pltpu.sync_copy(
                        zer, agg.at[pl.ds(ck * CH, CH)]))
                    plsc.subcore_barrier()

                    def step(kk, carry, h_hbm=h_hbm):
                        for t in range(PIPE):
                            j = kk * PIPE + t
                            @pl.when(j < NB)
                            def _(j=j, t=t):
                                base = sid * (NB * EB) + j * EB
                                pltpu.make_async_copy(
                                    src_hbm.at[pl.ds(base, EB)], sidx.at[t],
                                    sem_is[t]).start()
                                pltpu.make_async_copy(
                                    dst_hbm.at[pl.ds(base, EB)], didx.at[t],
                                    sem_id[t]).start()
                        for t in range(PIPE):
                            j = kk * PIPE + t
                            @pl.when(j < NB)
                            def _(j=j, t=t):
                                base = sid * (NB * EB) + j * EB
                                pltpu.make_async_copy(
                                    src_hbm.at[pl.ds(base, EB)], sidx.at[t],
                                    sem_is[t]).wait()
                                pltpu.make_async_copy(
                                    h_hbm.at[sidx.at[t]], rows.at[t],
                                    sem_g[t]).start()
                        for t in range(PIPE):
                            j = kk * PIPE + t
                            @pl.when(j < NB)
                            def _(j=j, t=t):
                                base = sid * (NB * EB) + j * EB
                                pltpu.make_async_copy(
                                    dst_hbm.at[pl.ds(base, EB)], didx.at[t],
                                    sem_id[t]).wait()
                                pltpu.make_async_copy(
                                    h_hbm.at[sidx.at[t]], rows.at[t],
                                    sem_g[t]).wait()
                                pltpu.make_async_copy(
                                    rows.at[t], agg.at[didx.at[t]],
                                    sem_s[t]).start(add=True)
                        for t in range(PIPE):
                            j = kk * PIPE + t
                            @pl.when(j < NB)
                            def _(j=j, t=t):
                                pltpu.make_async_copy(
                                    rows.at[t], agg.at[didx.at[t]],
                                    sem_s[t]).wait()
                        return carry

                    lax.fori_loop(0, (NB + PIPE - 1) // PIPE, step, 0)
                    plsc.subcore_barrier()

                    def wback(ck, out_hbm=out_hbm):
                        sl = pl.ds(ck * CH, CH)
                        pltpu.sync_copy(agg.at[sl], zer)
                        pltpu.sync_copy(zer, out_hbm.at[sl])

                    _chunked(sid, wback)
                    plsc.subcore_barrier()
                    if k + 1 < C:
                        pltpu.sync_copy(zeros_hbm, zer)

    def call(src1, dst1, zeros_buf, h_list):
        f = pl.kernel(
            body,
            out_type=[jax.ShapeDtypeStruct((N_N, W), F32)] * n_tab,
            mesh=_MESH,
            scratch_types=[
                pltpu.VMEM((PIPE, EB), jnp.int32),
                pltpu.VMEM((PIPE, EB), jnp.int32),
                pltpu.VMEM((PIPE, EB, W), F32),
                pltpu.VMEM((CH, W), F32),
                pltpu.VMEM_SHARED((N_N, W), F32),
            ] + [pltpu.SemaphoreType.DMA] * (4 * PIPE),
        )
        return f(src1, dst1, zeros_buf, *h_list)

    return call


_spmm_128x2 = _make_spmm(2, 128)
_spmm_128x1 = _make_spmm(1, 128)


EB3 = 40
NB3 = 125


def _spmm_split_body(src_hbm, dst_hbm, zeros_hbm, h_hbm, out0, out1,
                     sidx, didx, rows, zer, agg, *sems):
    sem_is, sem_id = sems[:SPIPE], sems[SPIPE:2 * SPIPE]
    sem_g, sem_s = sems[2 * SPIPE:3 * SPIPE], sems[3 * SPIPE:]
    cid = lax.axis_index("c")
    sid = lax.axis_index("s")
    pltpu.sync_copy(zeros_hbm, zer)
    for side in range(2):
        @pl.when(cid == side)
        def _(side=side):
            out_hbm = out0 if side == 0 else out1
            _chunked(sid, lambda ck: pltpu.sync_copy(
                zer, agg.at[pl.ds(ck * CH, CH)]))
            plsc.subcore_barrier()

            def step(kk, carry):
                for t in range(SPIPE):
                    j = kk * SPIPE + t
                    @pl.when(j < NB3)
                    def _(j=j, t=t):
                        base = side * (N_E // 2) + sid * (NB3 * EB3) + j * EB3
                        pltpu.make_async_copy(
                            src_hbm.at[pl.ds(base, EB3)], sidx.at[t],
                            sem_is[t]).start()
                        pltpu.make_async_copy(
                            dst_hbm.at[pl.ds(base, EB3)], didx.at[t],
                            sem_id[t]).start()
                for t in range(SPIPE):
                    j = kk * SPIPE + t
                    @pl.when(j < NB3)
                    def _(j=j, t=t):
                        base = side * (N_E // 2) + sid * (NB3 * EB3) + j * EB3
                        pltpu.make_async_copy(
                            src_hbm.at[pl.ds(base, EB3)], sidx.at[t],
                            sem_is[t]).wait()
                        pltpu.make_async_copy(
                            h_hbm.at[sidx.at[t]], rows.at[t], sem_g[t]).start()
                for t in range(SPIPE):
                    j = kk * SPIPE + t
                    @pl.when(j < NB3)
                    def _(j=j, t=t):
                        base = side * (N_E // 2) + sid * (NB3 * EB3) + j * EB3
                        pltpu.make_async_copy(
                            dst_hbm.at[pl.ds(base, EB3)], didx.at[t],
                            sem_id[t]).wait()
                        pltpu.make_async_copy(
                            h_hbm.at[sidx.at[t]], rows.at[t], sem_g[t]).wait()
                        pltpu.make_async_copy(
                            rows.at[t], agg.at[didx.at[t]],
                            sem_s[t]).start(add=True)
                for t in range(SPIPE):
                    j = kk * SPIPE + t
                    @pl.when(j < NB3)
                    def _(j=j, t=t):
                        pltpu.make_async_copy(
                            rows.at[t], agg.at[didx.at[t]], sem_s[t]).wait()
                return carry

            lax.fori_loop(0, (NB3 + SPIPE - 1) // SPIPE, step, 0)
            plsc.subcore_barrier()

            def wback(ck, out_hbm=out_hbm):
                sl = pl.ds(ck * CH, CH)
                pltpu.sync_copy(agg.at[sl], zer)
                pltpu.sync_copy(zer, out_hbm.at[sl])

            _chunked(sid, wback)


def _spmm_split(src1, dst1, zeros_buf, h):
    f = pl.kernel(
        _spmm_split_body,
        out_type=[jax.ShapeDtypeStruct((N_N, 128), F32)] * 2,
        mesh=_MESH,
        scratch_types=[
            pltpu.VMEM((SPIPE, EB3), jnp.int32),
            pltpu.VMEM((SPIPE, EB3), jnp.int32),
            pltpu.VMEM((SPIPE, EB3, 128), F32),
            pltpu.VMEM((CH, 128), F32),
            pltpu.VMEM_SHARED((N_N, 128), F32),
        ] + [pltpu.SemaphoreType.DMA] * (4 * SPIPE),
    )
    return f(src1, dst1, zeros_buf, h)


EB2 = 40
NB2 = 125


def _edge_body(hs_hbm, hd_hbm, src_hbm, dst_hbm, out_hbm,
               sidx, didx, a_v, b_v, o_v, *sems):
    sem_is, sem_id = sems[:EP], sems[EP:2 * EP]
    sem_a, sem_b, sem_o = sems[2 * EP:3 * EP], sems[3 * EP:4 * EP], sems[4 * EP:]
    cid = lax.axis_index("c")
    sid = lax.axis_index("s")
    wid = sid * 2 + cid

    def step(kk, carry):
        for t in range(EP):
            j = kk * EP + t
            @pl.when(j < NB2)
            def _(j=j, t=t):
                base = wid * (NB2 * EB2) + j * EB2
                pltpu.make_async_copy(
                    src_hbm.at[pl.ds(base, EB2)], sidx.at[t], sem_is[t]).start()
                pltpu.make_async_copy(
                    dst_hbm.at[pl.ds(base, EB2)], didx.at[t], sem_id[t]).start()
        for t in range(EP):
            j = kk * EP + t
            @pl.when(j < NB2)
            def _(j=j, t=t):
                base = wid * (NB2 * EB2) + j * EB2
                pltpu.make_async_copy(
                    src_hbm.at[pl.ds(base, EB2)], sidx.at[t], sem_is[t]).wait()
                pltpu.make_async_copy(
                    dst_hbm.at[pl.ds(base, EB2)], didx.at[t], sem_id[t]).wait()
                pltpu.make_async_copy(
                    hs_hbm.at[sidx.at[t]], a_v.at[t], sem_a[t]).start()
                pltpu.make_async_copy(
                    hd_hbm.at[didx.at[t]], b_v.at[t], sem_b[t]).start()
        for t in range(EP):
            j = kk * EP + t
            @pl.when(j < NB2)
            def _(j=j, t=t):
                pltpu.make_async_copy(
                    hs_hbm.at[sidx.at[t]], a_v.at[t], sem_a[t]).wait()
                pltpu.make_async_copy(
                    hd_hbm.at[didx.at[t]], b_v.at[t], sem_b[t]).wait()
                for r in range(EB2):
                    o_v[t, r, :] = (a_v[t, r, pl.ds(0, 16)]
                                    + b_v[t, r, pl.ds(0, 16)])
                pltpu.make_async_copy(
                    o_v.at[t],
                    out_hbm.at[pl.ds(wid * (NB2 * EB2) + j * EB2, EB2)],
                    sem_o[t]).start()
        for t in range(EP):
            j = kk * EP + t
            @pl.when(j < NB2)
            def _(j=j, t=t):
                pltpu.make_async_copy(
                    o_v.at[t],
                    out_hbm.at[pl.ds(wid * (NB2 * EB2) + j * EB2, EB2)],
                    sem_o[t]).wait()
        return carry

    lax.fori_loop(0, (NB2 + EP - 1) // EP, step, 0)


def _edge_call(hs, hd, src1, dst1):
    f = pl.kernel(
        _edge_body,
        out_type=jax.ShapeDtypeStruct((N_E, 16), F32),
        mesh=_MESH,
        scratch_types=[
            pltpu.VMEM((EP, EB2), jnp.int32),
            pltpu.VMEM((EP, EB2), jnp.int32),
            pltpu.VMEM((EP, EB2, 128), F32),
            pltpu.VMEM((EP, EB2, 128), F32),
            pltpu.VMEM((EP, EB2, 16), F32),
        ] + [pltpu.SemaphoreType.DMA] * (5 * EP),
    )
    return f(hs, hd, src1, dst1)


BM = 2000
GRID = N_N // BM


def _norms(deg_ref):
    return lax.rsqrt(jnp.clip(deg_ref[...][:, :1], 1.0, None))


def _tc0_body(x_ref, wl_ref, bl_ref, w1_ref, dg_ref, *outs):
    t = jnp.dot(x_ref[...], wl_ref[...], preferred_element_type=F32) + bl_ref[...]
    p = jnp.dot(t, w1_ref[...], preferred_element_type=F32) * _norms(dg_ref)
    for i, o in enumerate(outs):
        o[...] = p[:, i * 128:(i + 1) * 128]


def _tc0(x, wlin, blin, w1, dego):
    return pl.pallas_call(
        _tc0_body,
        grid=(GRID,),
        in_specs=[
            pl.BlockSpec((BM, 256), lambda i: (i, 0)),
            pl.BlockSpec((256, 512), lambda i: (0, 0)),
            pl.BlockSpec((1, 512), lambda i: (0, 0)),
            pl.BlockSpec((512, 512), lambda i: (0, 0)),
            pl.BlockSpec((BM, 128), lambda i: (i, 0)),
        ],
        out_specs=[pl.BlockSpec((BM, 128), lambda i: (i, 0))] * 4,
        out_shape=[jax.ShapeDtypeStruct((N_N, 128), F32)] * 4,
    )(x, wlin, blin, w1, dego)


def _tc_mid(aggs, degi, dego, b, w_next, out_w, n_out, combine="concat"):
    if combine == "concat":
        in_w = sum(a.shape[1] for a in aggs)
    else:
        in_w = aggs[0].shape[1]
    f_out = w_next.shape[1]
    n_in = len(aggs)

    def body(*refs):
        agg_refs = refs[:n_in]
        di_ref, do_ref, b_ref, w_ref = refs[n_in:n_in + 4]
        outs = refs[n_in + 4:]
        if combine == "concat":
            agg = jnp.concatenate([r[...] for r in agg_refs], axis=1)
        else:
            agg = agg_refs[0][...]
            for r in agg_refs[1:]:
                agg = agg + r[...]
        y = jnp.maximum(agg * _norms(di_ref) + b_ref[...], 0.0)
        p = jnp.dot(y, w_ref[...], preferred_element_type=F32) * _norms(do_ref)
        for i, o in enumerate(outs):
            o[...] = p[:, i * out_w:(i + 1) * out_w]

    return pl.pallas_call(
        body,
        grid=(GRID,),
        in_specs=(
            [pl.BlockSpec((BM, a.shape[1]), lambda i: (i, 0)) for a in aggs]
            + [
                pl.BlockSpec((BM, 128), lambda i: (i, 0)),
                pl.BlockSpec((BM, 128), lambda i: (i, 0)),
                pl.BlockSpec((1, in_w), lambda i: (0, 0)),
                pl.BlockSpec((in_w, f_out), lambda i: (0, 0)),
            ]
        ),
        out_specs=[pl.BlockSpec((BM, out_w), lambda i: (i, 0))] * n_out,
        out_shape=[jax.ShapeDtypeStruct((N_N, out_w), F32)] * n_out,
    )(*aggs, degi, dego, b, w_next)


def _tc_last(aggs, degi, b4, w_top, w_bot, bcls):
    n_in = len(aggs)

    def body(*refs):
        agg_refs = refs[:n_in]
        di_ref, b_ref, wt_ref, wb_ref, bc_ref, hs_o, hd_o = refs[n_in:]
        agg = agg_refs[0][...]
        for r in agg_refs[1:]:
            agg = agg + r[...]
        y = jnp.maximum(agg * _norms(di_ref) + b_ref[...], 0.0)
        pad = jnp.zeros((BM, 112), F32)
        hs = jnp.dot(y, wt_ref[...], preferred_element_type=F32) + bc_ref[...]
        hd = jnp.dot(y, wb_ref[...], preferred_element_type=F32)
        hs_o[...] = jnp.concatenate([hs, pad], axis=1)
        hd_o[...] = jnp.concatenate([hd, pad], axis=1)

    return pl.pallas_call(
        body,
        grid=(GRID,),
        in_specs=(
            [pl.BlockSpec((BM, a.shape[1]), lambda i: (i, 0)) for a in aggs]
            + [
                pl.BlockSpec((BM, 128), lambda i: (i, 0)),
                pl.BlockSpec((1, 128), lambda i: (0, 0)),
                pl.BlockSpec((128, 16), lambda i: (0, 0)),
                pl.BlockSpec((128, 16), lambda i: (0, 0)),
                pl.BlockSpec((1, 16), lambda i: (0, 0)),
            ]
        ),
        out_specs=[pl.BlockSpec((BM, 128), lambda i: (i, 0))] * 2,
        out_shape=[jax.ShapeDtypeStruct((N_N, 128), F32)] * 2,
    )(*aggs, degi, b4, w_top, w_bot, bcls)


def _jax_deg(src, dst):
    dego = jnp.zeros((N_N,), F32).at[src].add(1.0)
    degi = jnp.zeros((N_N,), F32).at[dst].add(1.0)
    one = jnp.ones((1, 128), F32)
    return dego[:, None] * one, degi[:, None] * one


def _jax_spmm_chunks(src, dst, h_list):
    return [jnp.zeros((N_N, h.shape[1]), F32).at[dst].add(h[src]) for h in h_list]


def _jax_spmm_split(src, dst, h):
    half = N_E // 2
    p0 = jnp.zeros((N_N, 128), F32).at[dst[:half]].add(h[src[:half]])
    p1 = jnp.zeros((N_N, 128), F32).at[dst[half:]].add(h[src[half:]])
    return [p0, p1]


def _jax_edge(hs, hd, src, dst):
    return (hs[src] + hd[dst])[:, :16]


def kernel(node_features, edge_index, edge_features, W_lin, b_lin,
           W1, b1, W2, b2, W3, b3, W4, b4, W_cls, b_cls):
    del edge_features
    src = edge_index[0]
    dst = edge_index[1]
    z128 = jnp.zeros((CH, 128), F32)
    ones128 = jnp.ones((EB, 128), F32)

    dego, degi = _deg_call(src, dst, ones128, z128)

    p1 = _tc0(node_features, W_lin, b_lin.reshape(1, 512), W1, dego)
    a1 = _spmm_128x2(src, dst, z128, p1)
    p2 = _tc_mid(a1, degi, dego, b1.reshape(1, 512), W2, 128, 2)
    a2 = _spmm_128x1(src, dst, z128, p2)
    p3 = _tc_mid(a2, degi, dego, b2.reshape(1, 256), W3, 128, 1)
    a3 = _spmm_split(src, dst, z128, p3[0])
    p4 = _tc_mid(a3, degi, dego, b3.reshape(1, 128), W4, 128, 1, combine="sum")
    a4 = _spmm_split(src, dst, z128, p4[0])
    hs, hd = _tc_last(a4, degi, b4.reshape(1, 128), W_cls[:128], W_cls[128:],
                      b_cls.reshape(1, 16))
    return _edge_call(hs, hd, src, dst)


# speedup vs baseline: 6.1725x; 1.0002x over previous
import jax
import jax.numpy as jnp
from jax import lax
from jax.experimental import pallas as pl
from jax.experimental.pallas import tpu as pltpu
from jax.experimental.pallas import tpu_sc as plsc

N_N = 10000
N_E = 160000
EB = 80
NB = 125
CH = 40
NCH = N_N // CH
F32 = jnp.float32

def _mesh():
    return plsc.VectorSubcoreMesh(core_axis_name="c", subcore_axis_name="s")


def _chunked(sid, fn):
    for z in range(16):
        ck = sid + 16 * z
        @pl.when(ck < NCH)
        def _(ck=ck):
            fn(ck)


PIPE = 3
DPIPE = 3
SPIPE = 3
EP = 2


def _deg_body(src_hbm, dst_hbm, ones_hbm, zeros_hbm, out0, out1,
              idx, ones_v, zer, deg_sh, *sems):
    sem_i = sems[:DPIPE]
    sem_s = sems[DPIPE:]
    cid = lax.axis_index("c")
    sid = lax.axis_index("s")
    pltpu.sync_copy(ones_hbm, ones_v)
    pltpu.sync_copy(zeros_hbm, zer)
    for side in range(2):
        @pl.when(cid == side)
        def _(side=side):
            e_hbm = src_hbm if side == 0 else dst_hbm
            out_hbm = out0 if side == 0 else out1
            _chunked(sid, lambda ck: pltpu.sync_copy(
                zer, deg_sh.at[pl.ds(ck * CH, CH)]))
            plsc.subcore_barrier()

            def step(k, carry):
                for t in range(DPIPE):
                    j = k * DPIPE + t
                    @pl.when(j < NB)
                    def _(j=j, t=t):
                        base = sid * (NB * EB) + j * EB
                        pltpu.make_async_copy(
                            e_hbm.at[pl.ds(base, EB)], idx.at[t], sem_i[t]).start()
                for t in range(DPIPE):
                    j = k * DPIPE + t
                    @pl.when(j < NB)
                    def _(j=j, t=t):
                        base = sid * (NB * EB) + j * EB
                        pltpu.make_async_copy(
                            e_hbm.at[pl.ds(base, EB)], idx.at[t], sem_i[t]).wait()
                        pltpu.make_async_copy(
                            ones_v, deg_sh.at[idx.at[t]], sem_s[t]).start(add=True)
                for t in range(DPIPE):
                    j = k * DPIPE + t
                    @pl.when(j < NB)
                    def _(j=j, t=t):
                        pltpu.make_async_copy(
                            ones_v, deg_sh.at[idx.at[t]], sem_s[t]).wait()
                return carry

            lax.fori_loop(0, (NB + DPIPE - 1) // DPIPE, step, 0)
            plsc.subcore_barrier()

            def wback(ck):
                sl = pl.ds(ck * CH, CH)
                pltpu.sync_copy(deg_sh.at[sl], zer)
                pltpu.sync_copy(zer, out_hbm.at[sl])

            _chunked(sid, wback)


def _deg_call(src1, dst1, ones128, zeros128):
    f = pl.kernel(
        _deg_body,
        out_type=[jax.ShapeDtypeStruct((N_N, 128), F32)] * 2,
        mesh=_mesh(),
        scratch_types=[
            pltpu.VMEM((DPIPE, EB), jnp.int32),
            pltpu.VMEM((EB, 128), F32),
            pltpu.VMEM((CH, 128), F32),
            pltpu.VMEM_SHARED((N_N, 128), F32),
        ] + [pltpu.SemaphoreType.DMA] * (2 * DPIPE),
    )
    return f(src1, dst1, ones128, zeros128)


def _make_spmm(n_chunks_per_core, width):
    C, W = n_chunks_per_core, width
    n_tab = 2 * C

    def body(src_hbm, dst_hbm, zeros_hbm, *rest):
        h_refs = rest[:n_tab]
        out_refs = rest[n_tab:2 * n_tab]
        sidx, didx, rows, zer, agg = rest[2 * n_tab:2 * n_tab + 5]
        sems = rest[2 * n_tab + 5:]
        sem_is, sem_id = sems[:PIPE], sems[PIPE:2 * PIPE]
        sem_g, sem_s = sems[2 * PIPE:3 * PIPE], sems[3 * PIPE:]
        cid = lax.axis_index("c")
        sid = lax.axis_index("s")
        pltpu.sync_copy(zeros_hbm, zer)
        for side in range(2):
            @pl.when(cid == side)
            def _(side=side):
                for k in range(C):
                    h_hbm = h_refs[side * C + k]
                    out_hbm = out_refs[side * C + k]
                    _chunked(sid, lambda ck: pltpu.sync_copy(
                        zer, agg.at[pl.ds(ck * CH, CH)]))
                    plsc.subcore_barrier()

                    def step(kk, carry, h_hbm=h_hbm):
                        for t in range(PIPE):
                            j = kk * PIPE + t
                            @pl.when(j < NB)
                            def _(j=j, t=t):
                                base = sid * (NB * EB) + j * EB
                                pltpu.make_async_copy(
                                    src_hbm.at[pl.ds(base, EB)], sidx.at[t],
                                    sem_is[t]).start()
                                pltpu.make_async_copy(
                                    dst_hbm.at[pl.ds(base, EB)], didx.at[t],
                                    sem_id[t]).start()
                        for t in range(PIPE):
                            j = kk * PIPE + t
                            @pl.when(j < NB)
                            def _(j=j, t=t):
                                base = sid * (NB * EB) + j * EB
                                pltpu.make_async_copy(
                                    src_hbm.at[pl.ds(base, EB)], sidx.at[t],
                                    sem_is[t]).wait()
                                pltpu.make_async_copy(
                                    h_hbm.at[sidx.at[t]], rows.at[t],
                                    sem_g[t]).start()
                        for t in range(PIPE):
                            j = kk * PIPE + t
                            @pl.when(j < NB)
                            def _(j=j, t=t):
                                base = sid * (NB * EB) + j * EB
                                pltpu.make_async_copy(
                                    dst_hbm.at[pl.ds(base, EB)], didx.at[t],
                                    sem_id[t]).wait()
                                pltpu.make_async_copy(
                                    h_hbm.at[sidx.at[t]], rows.at[t],
                                    sem_g[t]).wait()
                                pltpu.make_async_copy(
                                    rows.at[t], agg.at[didx.at[t]],
                                    sem_s[t]).start(add=True)
                        for t in range(PIPE):
                            j = kk * PIPE + t
                            @pl.when(j < NB)
                            def _(j=j, t=t):
                                pltpu.make_async_copy(
                                    rows.at[t], agg.at[didx.at[t]],
                                    sem_s[t]).wait()
                        return carry

                    lax.fori_loop(0, (NB + PIPE - 1) // PIPE, step, 0)
                    plsc.subcore_barrier()

                    def wback(ck, out_hbm=out_hbm):
                        sl = pl.ds(ck * CH, CH)
                        pltpu.sync_copy(agg.at[sl], zer)
                        pltpu.sync_copy(zer, out_hbm.at[sl])

                    _chunked(sid, wback)
                    plsc.subcore_barrier()
                    if k + 1 < C:
                        pltpu.sync_copy(zeros_hbm, zer)

    def call(src1, dst1, zeros_buf, h_list):
        f = pl.kernel(
            body,
            out_type=[jax.ShapeDtypeStruct((N_N, W), F32)] * n_tab,
            mesh=_mesh(),
            scratch_types=[
                pltpu.VMEM((PIPE, EB), jnp.int32),
                pltpu.VMEM((PIPE, EB), jnp.int32),
                pltpu.VMEM((PIPE, EB, W), F32),
                pltpu.VMEM((CH, W), F32),
                pltpu.VMEM_SHARED((N_N, W), F32),
            ] + [pltpu.SemaphoreType.DMA] * (4 * PIPE),
        )
        return f(src1, dst1, zeros_buf, *h_list)

    return call


_spmm_128x2 = _make_spmm(2, 128)
_spmm_128x1 = _make_spmm(1, 128)


EB3 = 40
NB3 = 125


def _spmm_split_body(src_hbm, dst_hbm, zeros_hbm, h_hbm, out0, out1,
                     sidx, didx, rows, zer, agg, *sems):
    sem_is, sem_id = sems[:SPIPE], sems[SPIPE:2 * SPIPE]
    sem_g, sem_s = sems[2 * SPIPE:3 * SPIPE], sems[3 * SPIPE:]
    cid = lax.axis_index("c")
    sid = lax.axis_index("s")
    pltpu.sync_copy(zeros_hbm, zer)
    for side in range(2):
        @pl.when(cid == side)
        def _(side=side):
            out_hbm = out0 if side == 0 else out1
            _chunked(sid, lambda ck: pltpu.sync_copy(
                zer, agg.at[pl.ds(ck * CH, CH)]))
            plsc.subcore_barrier()

            def step(kk, carry):
                for t in range(SPIPE):
                    j = kk * SPIPE + t
                    @pl.when(j < NB3)
                    def _(j=j, t=t):
                        base = side * (N_E // 2) + sid * (NB3 * EB3) + j * EB3
                        pltpu.make_async_copy(
                            src_hbm.at[pl.ds(base, EB3)], sidx.at[t],
                            sem_is[t]).start()
                        pltpu.make_async_copy(
                            dst_hbm.at[pl.ds(base, EB3)], didx.at[t],
                            sem_id[t]).start()
                for t in range(SPIPE):
                    j = kk * SPIPE + t
                    @pl.when(j < NB3)
                    def _(j=j, t=t):
                        base = side * (N_E // 2) + sid * (NB3 * EB3) + j * EB3
                        pltpu.make_async_copy(
                            src_hbm.at[pl.ds(base, EB3)], sidx.at[t],
                            sem_is[t]).wait()
                        pltpu.make_async_copy(
                            h_hbm.at[sidx.at[t]], rows.at[t], sem_g[t]).start()
                for t in range(SPIPE):
                    j = kk * SPIPE + t
                    @pl.when(j < NB3)
                    def _(j=j, t=t):
                        base = side * (N_E // 2) + sid * (NB3 * EB3) + j * EB3
                        pltpu.make_async_copy(
                            dst_hbm.at[pl.ds(base, EB3)], didx.at[t],
                            sem_id[t]).wait()
                        pltpu.make_async_copy(
                            h_hbm.at[sidx.at[t]], rows.at[t], sem_g[t]).wait()
                        pltpu.make_async_copy(
                            rows.at[t], agg.at[didx.at[t]],
                            sem_s[t]).start(add=True)
                for t in range(SPIPE):
                    j = kk * SPIPE + t
                    @pl.when(j < NB3)
                    def _(j=j, t=t):
                        pltpu.make_async_copy(
                            rows.at[t], agg.at[didx.at[t]], sem_s[t]).wait()
                return carry

            lax.fori_loop(0, (NB3 + SPIPE - 1) // SPIPE, step, 0)
            plsc.subcore_barrier()

            def wback(ck, out_hbm=out_hbm):
                sl = pl.ds(ck * CH, CH)
                pltpu.sync_copy(agg.at[sl], zer)
                pltpu.sync_copy(zer, out_hbm.at[sl])

            _chunked(sid, wback)


def _spmm_split(src1, dst1, zeros_buf, h):
    f = pl.kernel(
        _spmm_split_body,
        out_type=[jax.ShapeDtypeStruct((N_N, 128), F32)] * 2,
        mesh=_mesh(),
        scratch_types=[
            pltpu.VMEM((SPIPE, EB3), jnp.int32),
            pltpu.VMEM((SPIPE, EB3), jnp.int32),
            pltpu.VMEM((SPIPE, EB3, 128), F32),
            pltpu.VMEM((CH, 128), F32),
            pltpu.VMEM_SHARED((N_N, 128), F32),
        ] + [pltpu.SemaphoreType.DMA] * (4 * SPIPE),
    )
    return f(src1, dst1, zeros_buf, h)


EB2 = 40
NB2 = 125


def _edge_body(hs_hbm, hd_hbm, src_hbm, dst_hbm, out_hbm,
               sidx, didx, a_v, b_v, o_v, *sems):
    sem_is, sem_id = sems[:EP], sems[EP:2 * EP]
    sem_a, sem_b, sem_o = sems[2 * EP:3 * EP], sems[3 * EP:4 * EP], sems[4 * EP:]
    cid = lax.axis_index("c")
    sid = lax.axis_index("s")
    wid = sid * 2 + cid

    def step(kk, carry):
        for t in range(EP):
            j = kk * EP + t
            @pl.when(j < NB2)
            def _(j=j, t=t):
                base = wid * (NB2 * EB2) + j * EB2
                pltpu.make_async_copy(
                    src_hbm.at[pl.ds(base, EB2)], sidx.at[t], sem_is[t]).start()
                pltpu.make_async_copy(
                    dst_hbm.at[pl.ds(base, EB2)], didx.at[t], sem_id[t]).start()
        for t in range(EP):
            j = kk * EP + t
            @pl.when(j < NB2)
            def _(j=j, t=t):
                base = wid * (NB2 * EB2) + j * EB2
                pltpu.make_async_copy(
                    src_hbm.at[pl.ds(base, EB2)], sidx.at[t], sem_is[t]).wait()
                pltpu.make_async_copy(
                    dst_hbm.at[pl.ds(base, EB2)], didx.at[t], sem_id[t]).wait()
                pltpu.make_async_copy(
                    hs_hbm.at[sidx.at[t]], a_v.at[t], sem_a[t]).start()
                pltpu.make_async_copy(
                    hd_hbm.at[didx.at[t]], b_v.at[t], sem_b[t]).start()
        for t in range(EP):
            j = kk * EP + t
            @pl.when(j < NB2)
            def _(j=j, t=t):
                pltpu.make_async_copy(
                    hs_hbm.at[sidx.at[t]], a_v.at[t], sem_a[t]).wait()
                pltpu.make_async_copy(
                    hd_hbm.at[didx.at[t]], b_v.at[t], sem_b[t]).wait()
                for r in range(EB2):
                    o_v[t, r, :] = (a_v[t, r, pl.ds(0, 16)]
                                    + b_v[t, r, pl.ds(0, 16)])
                pltpu.make_async_copy(
                    o_v.at[t],
                    out_hbm.at[pl.ds(wid * (NB2 * EB2) + j * EB2, EB2)],
                    sem_o[t]).start()
        for t in range(EP):
            j = kk * EP + t
            @pl.when(j < NB2)
            def _(j=j, t=t):
                pltpu.make_async_copy(
                    o_v.at[t],
                    out_hbm.at[pl.ds(wid * (NB2 * EB2) + j * EB2, EB2)],
                    sem_o[t]).wait()
        return carry

    lax.fori_loop(0, (NB2 + EP - 1) // EP, step, 0)


def _edge_call(hs, hd, src1, dst1):
    f = pl.kernel(
        _edge_body,
        out_type=jax.ShapeDtypeStruct((N_E, 16), F32),
        mesh=_mesh(),
        scratch_types=[
            pltpu.VMEM((EP, EB2), jnp.int32),
            pltpu.VMEM((EP, EB2), jnp.int32),
            pltpu.VMEM((EP, EB2, 128), F32),
            pltpu.VMEM((EP, EB2, 128), F32),
            pltpu.VMEM((EP, EB2, 16), F32),
        ] + [pltpu.SemaphoreType.DMA] * (5 * EP),
    )
    return f(hs, hd, src1, dst1)


BM = 2000
GRID = N_N // BM


def _norms(deg_ref):
    return lax.rsqrt(jnp.clip(deg_ref[...][:, :1], 1.0, None))


def _tc0_body(x_ref, wl_ref, bl_ref, w1_ref, dg_ref, *outs):
    t = jnp.dot(x_ref[...], wl_ref[...], preferred_element_type=F32) + bl_ref[...]
    p = jnp.dot(t, w1_ref[...], preferred_element_type=F32) * _norms(dg_ref)
    for i, o in enumerate(outs):
        o[...] = p[:, i * 128:(i + 1) * 128]


def _tc0(x, wlin, blin, w1, dego):
    return pl.pallas_call(
        _tc0_body,
        grid=(GRID,),
        in_specs=[
            pl.BlockSpec((BM, 256), lambda i: (i, 0)),
            pl.BlockSpec((256, 512), lambda i: (0, 0)),
            pl.BlockSpec((1, 512), lambda i: (0, 0)),
            pl.BlockSpec((512, 512), lambda i: (0, 0)),
            pl.BlockSpec((BM, 128), lambda i: (i, 0)),
        ],
        out_specs=[pl.BlockSpec((BM, 128), lambda i: (i, 0))] * 4,
        out_shape=[jax.ShapeDtypeStruct((N_N, 128), F32)] * 4,
    )(x, wlin, blin, w1, dego)


def _tc_mid(aggs, degi, dego, b, w_next, out_w, n_out, combine="concat"):
    if combine == "concat":
        in_w = sum(a.shape[1] for a in aggs)
    else:
        in_w = aggs[0].shape[1]
    f_out = w_next.shape[1]
    n_in = len(aggs)

    def body(*refs):
        agg_refs = refs[:n_in]
        di_ref, do_ref, b_ref, w_ref = refs[n_in:n_in + 4]
        outs = refs[n_in + 4:]
        if combine == "concat":
            agg = jnp.concatenate([r[...] for r in agg_refs], axis=1)
        else:
            agg = agg_refs[0][...]
            for r in agg_refs[1:]:
                agg = agg + r[...]
        y = jnp.maximum(agg * _norms(di_ref) + b_ref[...], 0.0)
        p = jnp.dot(y, w_ref[...], preferred_element_type=F32) * _norms(do_ref)
        for i, o in enumerate(outs):
            o[...] = p[:, i * out_w:(i + 1) * out_w]

    return pl.pallas_call(
        body,
        grid=(GRID,),
        in_specs=(
            [pl.BlockSpec((BM, a.shape[1]), lambda i: (i, 0)) for a in aggs]
            + [
                pl.BlockSpec((BM, 128), lambda i: (i, 0)),
                pl.BlockSpec((BM, 128), lambda i: (i, 0)),
                pl.BlockSpec((1, in_w), lambda i: (0, 0)),
                pl.BlockSpec((in_w, f_out), lambda i: (0, 0)),
            ]
        ),
        out_specs=[pl.BlockSpec((BM, out_w), lambda i: (i, 0))] * n_out,
        out_shape=[jax.ShapeDtypeStruct((N_N, out_w), F32)] * n_out,
    )(*aggs, degi, dego, b, w_next)


def _tc_last(aggs, degi, b4, w_top, w_bot, bcls):
    n_in = len(aggs)

    def body(*refs):
        agg_refs = refs[:n_in]
        di_ref, b_ref, wt_ref, wb_ref, bc_ref, hs_o, hd_o = refs[n_in:]
        agg = agg_refs[0][...]
        for r in agg_refs[1:]:
            agg = agg + r[...]
        y = jnp.maximum(agg * _norms(di_ref) + b_ref[...], 0.0)
        pad = jnp.zeros((BM, 112), F32)
        hs = jnp.dot(y, wt_ref[...], preferred_element_type=F32) + bc_ref[...]
        hd = jnp.dot(y, wb_ref[...], preferred_element_type=F32)
        hs_o[...] = jnp.concatenate([hs, pad], axis=1)
        hd_o[...] = jnp.concatenate([hd, pad], axis=1)

    return pl.pallas_call(
        body,
        grid=(GRID,),
        in_specs=(
            [pl.BlockSpec((BM, a.shape[1]), lambda i: (i, 0)) for a in aggs]
            + [
                pl.BlockSpec((BM, 128), lambda i: (i, 0)),
                pl.BlockSpec((1, 128), lambda i: (0, 0)),
                pl.BlockSpec((128, 16), lambda i: (0, 0)),
                pl.BlockSpec((128, 16), lambda i: (0, 0)),
                pl.BlockSpec((1, 16), lambda i: (0, 0)),
            ]
        ),
        out_specs=[pl.BlockSpec((BM, 128), lambda i: (i, 0))] * 2,
        out_shape=[jax.ShapeDtypeStruct((N_N, 128), F32)] * 2,
    )(*aggs, degi, b4, w_top, w_bot, bcls)


def kernel(node_features, edge_index, edge_features, W_lin, b_lin,
           W1, b1, W2, b2, W3, b3, W4, b4, W_cls, b_cls):
    del edge_features
    src = edge_index[0]
    dst = edge_index[1]
    z128 = jnp.zeros((CH, 128), F32)
    ones128 = jnp.ones((EB, 128), F32)

    dego, degi = _deg_call(src, dst, ones128, z128)

    p1 = _tc0(node_features, W_lin, b_lin.reshape(1, 512), W1, dego)
    a1 = _spmm_128x2(src, dst, z128, p1)
    p2 = _tc_mid(a1, degi, dego, b1.reshape(1, 512), W2, 128, 2)
    a2 = _spmm_128x1(src, dst, z128, p2)
    p3 = _tc_mid(a2, degi, dego, b2.reshape(1, 256), W3, 128, 1)
    a3 = _spmm_split(src, dst, z128, p3[0])
    p4 = _tc_mid(a3, degi, dego, b3.reshape(1, 128), W4, 128, 1, combine="sum")
    a4 = _spmm_split(src, dst, z128, p4[0])
    hs, hd = _tc_last(a4, degi, b4.reshape(1, 128), W_cls[:128], W_cls[128:],
                      b_cls.reshape(1, 16))
    return _edge_call(hs, hd, src, dst)
